# Initial kernel scaffold; baseline (speedup 1.0000x reference)
import jax
import jax.numpy as jnp
from jax.experimental import pallas as pl


def kernel(feat, edge_index, W1, b1, W2, b2, eps):
    raise NotImplementedError("write your pallas kernel here")


# speedup vs baseline: 3.1505x; 3.1505x over previous
import functools

import jax
import jax.numpy as jnp
from jax import lax
from jax.experimental import pallas as pl
from jax.experimental.pallas import tpu as pltpu
from jax.experimental.pallas import tpu_sc as plsc

_NSUB = 16
_CH = 128


def _sc_segment_sum(feat_t, src2, dst2, zeros_blk, *, n_pad, n_chunks):
    cps = n_chunks // _NSUB
    zr = n_pad // _NSUB
    mesh = plsc.VectorSubcoreMesh(core_axis_name="c", subcore_axis_name="s")

    @functools.partial(
        pl.kernel,
        out_type=jax.ShapeDtypeStruct((2 * n_pad, 128), jnp.float32),
        mesh=mesh,
        scratch_types=[
            pltpu.VMEM((_CH,), jnp.int32),
            pltpu.VMEM((_CH,), jnp.int32),
            pltpu.VMEM((_CH, 128), jnp.float32),
            pltpu.VMEM_SHARED((n_pad, 128), jnp.float32),
        ],
    )
    def body(feat_hbm, src_hbm, dst_hbm, zero_hbm, out_hbm, srcv, dstv, rows, acc):
        c = lax.axis_index("c")
        s = lax.axis_index("s")
        row0 = s * zr
        pltpu.sync_copy(zero_hbm, acc.at[pl.ds(row0, zr)])
        plsc.subcore_barrier()
        offs = c * n_pad

        @pl.loop(0, cps)
        def _(k):
            cid = s * cps + k
            pltpu.sync_copy(src_hbm.at[cid], srcv)
            pltpu.sync_copy(dst_hbm.at[cid], dstv)
            for i in range(_CH // 16):
                sl = pl.ds(i * 16, 16)
                srcv[sl] = srcv[sl] + offs
            pltpu.sync_copy(feat_hbm.at[srcv], rows)
            pltpu.sync_copy(rows, acc.at[dstv], add=True)

        plsc.subcore_barrier()
        pltpu.sync_copy(acc.at[pl.ds(row0, zr)],
                        out_hbm.at[pl.ds(offs + row0, zr)])

    return body(feat_t, src2, dst2, zeros_blk)


def _tc_mlp(feat_p, neigh_flat, W1, b1, W2, b2, eps, *, n_pad, bm):
    d = feat_p.shape[1]
    h = W1.shape[1]
    nb = n_pad // bm

    def body(eps_sm, feat_r, nlo_r, nhi_r, w1_r, b1_r, w2_r, b2_r, out_r):
        scale = 1.0 + eps_sm[0]
        neigh = jnp.concatenate([nlo_r[...], nhi_r[...]], axis=1)
        rst = scale * feat_r[...] + neigh
        acts = jnp.maximum(
            jnp.dot(rst, w1_r[...], preferred_element_type=jnp.float32)
            + b1_r[...], 0.0)
        out_r[...] = (jnp.dot(acts, w2_r[...], preferred_element_type=jnp.float32)
                      + b2_r[...])

    return pl.pallas_call(
        body,
        grid=(nb,),
        in_specs=[
            pl.BlockSpec(memory_space=pltpu.SMEM),
            pl.BlockSpec((bm, d), lambda i: (i, 0)),
            pl.BlockSpec((bm, 128), lambda i: (i, 0)),
            pl.BlockSpec((bm, 128), lambda i: (i + nb, 0)),
            pl.BlockSpec((d, h), lambda i: (0, 0)),
            pl.BlockSpec((1, h), lambda i: (0, 0)),
            pl.BlockSpec((h, d), lambda i: (0, 0)),
            pl.BlockSpec((1, d), lambda i: (0, 0)),
        ],
        out_specs=pl.BlockSpec((bm, d), lambda i: (i, 0)),
        out_shape=jax.ShapeDtypeStruct((n_pad, d), jnp.float32),
        compiler_params=pltpu.CompilerParams(
            dimension_semantics=("arbitrary",)),
    )(eps, feat_p, neigh_flat, neigh_flat, W1,
      b1.reshape(1, h), W2, b2.reshape(1, d))


def kernel(feat, edge_index, W1, b1, W2, b2, eps):
    n, d = feat.shape
    e = edge_index.shape[1]
    bm = 512
    n_pad = ((n + bm - 1) // bm) * bm
    cps = -(-e // (_CH * _NSUB))
    n_chunks = cps * _NSUB
    e_pad = n_chunks * _CH

    src = edge_index[0]
    dst = edge_index[1]
    pad = e_pad - e
    src2 = jnp.concatenate([src, jnp.zeros((pad,), jnp.int32)]).reshape(n_chunks, _CH)
    dst2 = jnp.concatenate([dst, jnp.full((pad,), n, jnp.int32)]).reshape(n_chunks, _CH)

    feat_p = jnp.pad(feat, ((0, n_pad - n), (0, 0)))
    feat_t = feat_p.reshape(n_pad, 2, 128).transpose(1, 0, 2).reshape(2 * n_pad, 128)
    zeros_blk = jnp.zeros((n_pad // _NSUB, 128), jnp.float32)

    neigh_flat = _sc_segment_sum(feat_t, src2, dst2, zeros_blk,
                                 n_pad=n_pad, n_chunks=n_chunks)
    out_p = _tc_mlp(feat_p, neigh_flat, W1, b1, W2, b2, eps,
                    n_pad=n_pad, bm=bm)
    return out_p[:n]


# speedup vs baseline: 3.3798x; 1.0728x over previous
import functools

import jax
import jax.numpy as jnp
from jax import lax
from jax.experimental import pallas as pl
from jax.experimental.pallas import tpu as pltpu
from jax.experimental.pallas import tpu_sc as plsc

_NSUB = 16
_CH = 128


def _sc_segment_sum(feat2, edges3, zeros_blk, *, n_acc, cps):
    zr = n_acc // _NSUB
    mesh = plsc.VectorSubcoreMesh(core_axis_name="c", subcore_axis_name="s")

    @functools.partial(
        pl.kernel,
        out_type=jax.ShapeDtypeStruct((2 * n_acc, 128), jnp.float32),
        mesh=mesh,
        scratch_types=[
            pltpu.VMEM((2, 2, _CH), jnp.int32),
            pltpu.VMEM((2, _CH, 128), jnp.float32),
            pltpu.VMEM_SHARED((n_acc, 128), jnp.float32),
            pltpu.SemaphoreType.DMA,
            pltpu.SemaphoreType.DMA,
            pltpu.SemaphoreType.DMA,
            pltpu.SemaphoreType.DMA,
        ],
    )
    def body(feat_hbm, e_hbm, zero_hbm, out_hbm, idx, rows, acc,
             gsem0, gsem1, isem0, isem1):
        c = lax.axis_index("c")
        s = lax.axis_index("s")
        gsems = (gsem0, gsem1)
        isems = (isem0, isem1)
        row0 = s * zr
        base = s * cps

        def iload(b, ck):
            pltpu.async_copy(e_hbm.at[base + ck], idx.at[b], isems[b])

        def iwait(b, ck):
            pltpu.make_async_copy(e_hbm.at[base + ck], idx.at[b],
                                  isems[b]).wait()

        def fixup(b):
            srow = idx.at[b].at[0]
            for i in range(_CH // 16):
                sl = pl.ds(i * 16, 16)
                srow[sl] = srow[sl] + c

        def gstart(b, ck):
            pltpu.async_copy(feat_hbm.at[idx.at[b].at[0]], rows.at[b],
                             gsems[b])

        def gwait(b):
            pltpu.make_async_copy(feat_hbm.at[idx.at[b].at[0]], rows.at[b],
                                  gsems[b]).wait()

        iload(0, 0)
        iload(1, 1)
        pltpu.sync_copy(zero_hbm, acc.at[pl.ds(row0, zr)])
        iwait(0, 0)
        fixup(0)
        gstart(0, 0)
        plsc.subcore_barrier()

        @pl.loop(0, cps // 2)
        def _(j):
            k = 2 * j
            for b in range(2):
                ck = k + b
                b2 = 1 - b
                gwait(b)

                @pl.when(ck + 1 < cps)
                def _():
                    iwait(b2, ck + 1)
                    fixup(b2)
                    gstart(b2, ck + 1)

                pltpu.sync_copy(rows.at[b], acc.at[idx.at[b].at[1]],
                                add=True)

                @pl.when(ck + 2 < cps)
                def _():
                    iload(b, ck + 2)

        plsc.subcore_barrier()
        pltpu.sync_copy(acc.at[pl.ds(row0, zr)],
                        out_hbm.at[pl.ds(c * n_acc + row0, zr)])

    return body(feat2, edges3, zeros_blk)


def _tc_mlp(feat, neigh3, W1b, b1, W2b, b2, eps, *, bm):
    n, d = feat.shape
    h = W1b.shape[1]
    nb = n // bm

    def body(eps_sm, feat_r, n3_r, w1_r, b1_r, w2_r, b2_r, out_r):
        scale = 1.0 + eps_sm[0]
        neigh = jnp.concatenate([n3_r[0], n3_r[1]], axis=1)
        rst = scale * feat_r[...] + neigh
        acts = jnp.maximum(
            jnp.dot(rst.astype(jnp.bfloat16), w1_r[...],
                    preferred_element_type=jnp.float32) + b1_r[...], 0.0)
        out_r[...] = (jnp.dot(acts.astype(jnp.bfloat16), w2_r[...],
                              preferred_element_type=jnp.float32) + b2_r[...])

    return pl.pallas_call(
        body,
        grid=(nb,),
        in_specs=[
            pl.BlockSpec(memory_space=pltpu.SMEM),
            pl.BlockSpec((bm, d), lambda i: (i, 0)),
            pl.BlockSpec((2, bm, 128), lambda i: (0, i, 0)),
            pl.BlockSpec((d, h), lambda i: (0, 0)),
            pl.BlockSpec((1, h), lambda i: (0, 0)),
            pl.BlockSpec((h, d), lambda i: (0, 0)),
            pl.BlockSpec((1, d), lambda i: (0, 0)),
        ],
        out_specs=pl.BlockSpec((bm, d), lambda i: (i, 0)),
        out_shape=jax.ShapeDtypeStruct((n, d), jnp.float32),
        compiler_params=pltpu.CompilerParams(
            dimension_semantics=("arbitrary",)),
    )(eps, feat, neigh3, W1b,
      b1.reshape(1, h), W2b, b2.reshape(1, d))


def kernel(feat, edge_index, W1, b1, W2, b2, eps):
    n, d = feat.shape
    e = edge_index.shape[1]
    bm = 400
    n_acc = 10240
    cps = -(-e // (_CH * _NSUB))
    cps += cps % 2
    n_chunks = cps * _NSUB
    e_pad = n_chunks * _CH

    src = edge_index[0]
    dst = edge_index[1]
    pad = e_pad - e
    src2 = jnp.concatenate([src * 2, jnp.zeros((pad,), jnp.int32)])
    dst2 = jnp.concatenate([dst, jnp.full((pad,), n, jnp.int32)])
    edges3 = jnp.stack([src2.reshape(n_chunks, _CH),
                        dst2.reshape(n_chunks, _CH)], axis=1)

    feat2 = feat.reshape(2 * n, 128)
    zeros_blk = jnp.zeros((n_acc // _NSUB, 128), jnp.float32)

    neigh_flat = _sc_segment_sum(feat2, edges3, zeros_blk,
                                 n_acc=n_acc, cps=cps)
    return _tc_mlp(feat, neigh_flat.reshape(2, n_acc, 128),
                   W1.astype(jnp.bfloat16), b1,
                   W2.astype(jnp.bfloat16), b2, eps, bm=bm)


# speedup vs baseline: 7.0507x; 2.0861x over previous
import functools

import jax
import jax.numpy as jnp
from jax import lax
from jax.experimental import pallas as pl
from jax.experimental.pallas import tpu as pltpu
from jax.experimental.pallas import tpu_sc as plsc

_NSUB = 16
_CH = 128


def _sc_segment_sum(feat2, edge_index, zeros_blk, *, n_feat, n_acc, n_chunks):
    zr = n_acc // _NSUB
    cps_lo = n_chunks // _NSUB
    rem = n_chunks % _NSUB
    max_pairs = (cps_lo + 2) // 2
    mesh = plsc.VectorSubcoreMesh(core_axis_name="c", subcore_axis_name="s")

    @functools.partial(
        pl.kernel,
        out_type=jax.ShapeDtypeStruct((2 * n_acc, 128), jnp.float32),
        mesh=mesh,
        scratch_types=[
            pltpu.VMEM((2, 2, _CH), jnp.int32),
            pltpu.VMEM((2, _CH, 128), jnp.float32),
            pltpu.VMEM_SHARED((n_acc, 128), jnp.float32),
            pltpu.SemaphoreType.DMA,
            pltpu.SemaphoreType.DMA,
            pltpu.SemaphoreType.DMA,
            pltpu.SemaphoreType.DMA,
        ],
    )
    def body(feat_hbm, e_hbm, zero_hbm, out_hbm, idx, rows, acc,
             gsem0, gsem1, isem0, isem1):
        c = lax.axis_index("c")
        s = lax.axis_index("s")
        gsems = (gsem0, gsem1)
        isems = (isem0, isem1)
        row0 = s * zr
        cnt = cps_lo + jnp.where(s < rem, 1, 0)
        base = s * cps_lo + jnp.minimum(s, rem)

        def iload(b, ck):
            pltpu.async_copy(e_hbm.at[:, pl.ds((base + ck) * _CH, _CH)],
                             idx.at[b], isems[b])

        def iwait(b, ck):
            pltpu.make_async_copy(e_hbm.at[:, pl.ds((base + ck) * _CH, _CH)],
                                  idx.at[b], isems[b]).wait()

        tab = feat_hbm.at[:, pl.ds(c * 128, 128)]

        def gstart(b, ck):
            pltpu.async_copy(tab.at[idx.at[b].at[0]], rows.at[b],
                             gsems[b])

        def gwait(b):
            pltpu.make_async_copy(tab.at[idx.at[b].at[0]], rows.at[b],
                                  gsems[b]).wait()

        iload(0, 0)
        iload(1, 1)
        pltpu.sync_copy(zero_hbm, acc.at[pl.ds(row0, zr)])
        iwait(0, 0)

        gstart(0, 0)
        plsc.subcore_barrier()

        @pl.loop(0, max_pairs)
        def _(j):
            k = 2 * j
            for b in range(2):
                ck = k + b
                b2 = 1 - b

                @pl.when(ck < cnt)
                def _():
                    gwait(b)

                    @pl.when(ck + 1 < cnt)
                    def _():
                        iwait(b2, ck + 1)

                        gstart(b2, ck + 1)

                    pltpu.sync_copy(rows.at[b], acc.at[idx.at[b].at[1]],
                                    add=True)

                    @pl.when(ck + 2 < cnt)
                    def _():
                        iload(b, ck + 2)

        plsc.subcore_barrier()
        pltpu.sync_copy(acc.at[pl.ds(row0, zr)],
                        out_hbm.at[pl.ds(c * n_acc + row0, zr)])

    return body(feat2, edge_index, zeros_blk)


def _tc_mlp(feat, neigh3, W1b, b1, W2b, b2, eps, *, bm):
    n, d = feat.shape
    h = W1b.shape[1]
    nb = n // bm

    def body(eps_sm, feat_r, n3_r, w1_r, b1_r, w2_r, b2_r, out_r):
        scale = 1.0 + eps_sm[0]
        neigh = jnp.concatenate([n3_r[0], n3_r[1]], axis=1)
        rst = scale * feat_r[...].astype(jnp.float32) + neigh
        acts = jnp.maximum(
            jnp.dot(rst.astype(jnp.bfloat16), w1_r[...],
                    preferred_element_type=jnp.float32) + b1_r[...], 0.0)
        out_r[...] = (jnp.dot(acts.astype(jnp.bfloat16), w2_r[...],
                              preferred_element_type=jnp.float32) + b2_r[...])

    return pl.pallas_call(
        body,
        grid=(nb,),
        in_specs=[
            pl.BlockSpec(memory_space=pltpu.SMEM),
            pl.BlockSpec((bm, d), lambda i: (i, 0)),
            pl.BlockSpec((2, bm, 128), lambda i: (0, i, 0)),
            pl.BlockSpec((d, h), lambda i: (0, 0)),
            pl.BlockSpec((1, h), lambda i: (0, 0)),
            pl.BlockSpec((h, d), lambda i: (0, 0)),
            pl.BlockSpec((1, d), lambda i: (0, 0)),
        ],
        out_specs=pl.BlockSpec((bm, d), lambda i: (i, 0)),
        out_shape=jax.ShapeDtypeStruct((n, d), jnp.float32),
        compiler_params=pltpu.CompilerParams(
            dimension_semantics=("arbitrary",)),
    )(eps, feat, neigh3, W1b,
      b1.reshape(1, h), W2b, b2.reshape(1, d))


def kernel(feat, edge_index, W1, b1, W2, b2, eps):
    n, d = feat.shape
    e = edge_index.shape[1]
    bm = 1000
    n_acc = 10240
    n_chunks = e // _CH

    feat2 = feat
    zeros_blk = jnp.zeros((n_acc // _NSUB, 128), jnp.float32)

    neigh_flat = _sc_segment_sum(feat2, edge_index, zeros_blk,
                                 n_feat=n, n_acc=n_acc, n_chunks=n_chunks)
    return _tc_mlp(feat.astype(jnp.bfloat16), neigh_flat.reshape(2, n_acc, 128),
                   W1.astype(jnp.bfloat16), b1,
                   W2.astype(jnp.bfloat16), b2, eps, bm=bm)


# speedup vs baseline: 7.5314x; 1.0682x over previous
import functools

import jax
import jax.numpy as jnp
from jax import lax
from jax.experimental import pallas as pl
from jax.experimental.pallas import tpu as pltpu
from jax.experimental.pallas import tpu_sc as plsc

_NSUB = 16
_CH = 128


def _sc_segment_sum(feat2, edge_index, zeros_blk, *, n_feat, n_acc, n_chunks):
    zr = n_acc // _NSUB
    cps_lo = n_chunks // _NSUB
    rem = n_chunks % _NSUB
    max_pairs = (cps_lo + 2) // 2
    mesh = plsc.VectorSubcoreMesh(core_axis_name="c", subcore_axis_name="s")

    @functools.partial(
        pl.kernel,
        out_type=jax.ShapeDtypeStruct((2 * n_acc, 128), jnp.float32),
        mesh=mesh,
        scratch_types=[
            pltpu.VMEM((2, 2, _CH), jnp.int32),
            pltpu.VMEM((2, _CH, 128), jnp.float32),
            pltpu.VMEM_SHARED((n_acc, 128), jnp.float32),
            pltpu.SemaphoreType.DMA,
            pltpu.SemaphoreType.DMA,
            pltpu.SemaphoreType.DMA,
            pltpu.SemaphoreType.DMA,
        ],
    )
    def body(feat_hbm, e_hbm, zero_hbm, out_hbm, idx, rows, acc,
             gsem0, gsem1, isem0, isem1):
        c = lax.axis_index("c")
        s = lax.axis_index("s")
        gsems = (gsem0, gsem1)
        isems = (isem0, isem1)
        row0 = s * zr
        cnt = cps_lo + jnp.where(s < rem, 1, 0)
        base = s * cps_lo + jnp.minimum(s, rem)

        def iload(b, ck):
            pltpu.async_copy(e_hbm.at[:, pl.ds((base + ck) * _CH, _CH)],
                             idx.at[b], isems[b])

        def iwait(b, ck):
            pltpu.make_async_copy(e_hbm.at[:, pl.ds((base + ck) * _CH, _CH)],
                                  idx.at[b], isems[b]).wait()

        def fixup(b):
            srow = idx.at[b].at[0]
            for i in range(_CH // 16):
                sl = pl.ds(i * 16, 16)
                srow[sl] = srow[sl] * 2 + c

        def gstart(b, ck):
            pltpu.async_copy(feat_hbm.at[idx.at[b].at[0]], rows.at[b],
                             gsems[b])

        def gwait(b):
            pltpu.make_async_copy(feat_hbm.at[idx.at[b].at[0]], rows.at[b],
                                  gsems[b]).wait()

        iload(0, 0)
        iload(1, 1)
        pltpu.sync_copy(zero_hbm, acc.at[pl.ds(row0, zr)])
        iwait(0, 0)
        fixup(0)
        gstart(0, 0)
        plsc.subcore_barrier()

        @pl.loop(0, max_pairs)
        def _(j):
            k = 2 * j
            for b in range(2):
                ck = k + b
                b2 = 1 - b

                @pl.when(ck < cnt)
                def _():
                    gwait(b)

                    @pl.when(ck + 1 < cnt)
                    def _():
                        iwait(b2, ck + 1)
                        fixup(b2)
                        gstart(b2, ck + 1)

                    pltpu.sync_copy(rows.at[b], acc.at[idx.at[b].at[1]],
                                    add=True)

                    @pl.when(ck + 2 < cnt)
                    def _():
                        iload(b, ck + 2)

        plsc.subcore_barrier()
        pltpu.sync_copy(acc.at[pl.ds(row0, zr)],
                        out_hbm.at[pl.ds(c * n_acc + row0, zr)])

    return body(feat2, edge_index, zeros_blk)


def _tc_mlp(feat, neigh3, W1b, b1, W2b, b2, eps, *, bm):
    n, d = feat.shape
    h = W1b.shape[1]
    nb = n // bm

    def body(eps_sm, feat_r, n3_r, w1_r, b1_r, w2_r, b2_r, out_r):
        scale = 1.0 + eps_sm[0]
        neigh = jnp.concatenate([n3_r[0], n3_r[1]], axis=1)
        rst = scale * feat_r[...].astype(jnp.float32) + neigh
        acts = jnp.maximum(
            jnp.dot(rst.astype(jnp.bfloat16), w1_r[...],
                    preferred_element_type=jnp.float32) + b1_r[...], 0.0)
        out_r[...] = (jnp.dot(acts.astype(jnp.bfloat16), w2_r[...],
                              preferred_element_type=jnp.float32) + b2_r[...])

    return pl.pallas_call(
        body,
        grid=(nb,),
        in_specs=[
            pl.BlockSpec(memory_space=pltpu.SMEM),
            pl.BlockSpec((bm, d), lambda i: (i, 0)),
            pl.BlockSpec((2, bm, 128), lambda i: (0, i, 0)),
            pl.BlockSpec((d, h), lambda i: (0, 0)),
            pl.BlockSpec((1, h), lambda i: (0, 0)),
            pl.BlockSpec((h, d), lambda i: (0, 0)),
            pl.BlockSpec((1, d), lambda i: (0, 0)),
        ],
        out_specs=pl.BlockSpec((bm, d), lambda i: (i, 0)),
        out_shape=jax.ShapeDtypeStruct((n, d), jnp.float32),
        compiler_params=pltpu.CompilerParams(
            dimension_semantics=("arbitrary",)),
    )(eps, feat, neigh3, W1b,
      b1.reshape(1, h), W2b, b2.reshape(1, d))


def kernel(feat, edge_index, W1, b1, W2, b2, eps):
    n, d = feat.shape
    e = edge_index.shape[1]
    bm = 2000
    n_acc = 10240
    n_chunks = e // _CH

    feat2 = feat.reshape(2 * n, 128)
    zeros_blk = jnp.zeros((n_acc // _NSUB, 128), jnp.float32)

    neigh_flat = _sc_segment_sum(feat2, edge_index, zeros_blk,
                                 n_feat=n, n_acc=n_acc, n_chunks=n_chunks)
    return _tc_mlp(feat.astype(jnp.bfloat16), neigh_flat.reshape(2, n_acc, 128),
                   W1.astype(jnp.bfloat16), b1,
                   W2.astype(jnp.bfloat16), b2, eps, bm=bm)
